# baseline (speedup 1.0000x reference)
import functools

import jax
import jax.numpy as jnp
from jax import lax
from jax.experimental import pallas as pl
from jax.experimental.pallas import tpu as pltpu
from jax.experimental.pallas import tpu_sc as plsc

B, T, C, U = 16, 2048, 512, 256
BLANK = 0
DEN_SCALE = 1.0
NEG_INF = -1e30

TB = 64
NB = T // TB
G = 16
NCHUNK_O = U // 16
NCHUNK_E = U // 16 + 1
_GROUPS = (tuple(range(16, 8, -1)), tuple(range(8, 0, -1)), (0,))
ALEN = G + U + 16

_LOG_COEF = (
    -0.01720806024968624, 0.18497517704963684, -0.8555376529693604,
    2.2311506271362305, -3.648834466934204, 4.204533100128174,
    -2.0990748405456543,
)
_LN2 = 0.6931471805599453


def _polylog(s):
    bits = plsc.bitcast(s, jnp.int32)
    e = (bits >> 23) - 127
    m = plsc.bitcast((bits & 0x007FFFFF) | 0x3F800000, jnp.float32)
    acc = jnp.full_like(m, _LOG_COEF[0])
    for c in _LOG_COEF[1:]:
        acc = acc * m + c
    return e.astype(jnp.float32) * _LN2 + acc


def _sc_num_body(lp_hbm, tgt_hbm, skip_hbm, il_hbm, tl_hbm, out_hbm,
                 lp_buf, row_v, tgt_v, skip_v, il_v, tl_v, res_v,
                 aov, aos, aev, aes, sem0, sem1):
    wid = lax.axis_index("s") * 2 + lax.axis_index("c")

    @pl.when(wid < B)
    def _worker():
        b = wid
        iota = lax.iota(jnp.int32, 16)
        zeros = iota * 0
        neg = jnp.full((16,), NEG_INF, jnp.float32)
        ones = jnp.full((16,), 1.0, jnp.float32)

        pltpu.sync_copy(tgt_hbm.at[b], tgt_v)
        pltpu.sync_copy(skip_hbm.at[b], skip_v)
        pltpu.sync_copy(il_hbm.at[b], il_v)
        pltpu.sync_copy(tl_hbm.at[b], tl_v)
        pltpu.sync_copy(lp_hbm.at[b, 0], row_v)
        il = il_v[pl.ds(0, 16)][0]

        pltpu.make_async_copy(
            lp_hbm.at[b, pl.ds(0, TB), :], lp_buf.at[pl.ds(0, TB), :], sem0
        ).start()
        pltpu.make_async_copy(
            lp_hbm.at[b, pl.ds(TB, TB), :], lp_buf.at[pl.ds(TB, TB), :], sem1
        ).start()

        for cidx in range(ALEN // 16):
            aov[pl.ds(16 * cidx, 16)] = neg
            aev[pl.ds(16 * cidx, 16)] = neg
            aos[pl.ds(16 * cidx, 16)] = ones
            aes[pl.ds(16 * cidx, 16)] = ones
        blank0 = plsc.load_gather(row_v, [zeros])
        tgt0 = plsc.load_gather(row_v, [tgt_v[pl.ds(0, 16)]])
        first = iota == 0
        aev[pl.ds(G, 16)] = jnp.where(first, blank0, neg)
        aov[pl.ds(G, 16)] = jnp.where(first, tgt0, neg)

        def one_step(t, k, kb):
            trow = (t - k * TB) + kb * TB
            trows = zeros + trow
            blankv = plsc.load_gather(lp_buf, [trows, zeros])
            for grp in _GROUPS:
                ld = []
                for ci in grp:
                    off = 16 * ci
                    idx = iota + (G - 1 + off)
                    shv = plsc.load_gather(aov, [idx])
                    shs = plsc.load_gather(aos, [idx])
                    e0v = aev[pl.ds(G + off, 16)]
                    e0s = aes[pl.ds(G + off, 16)]
                    if ci < NCHUNK_O:
                        a0v = aov[pl.ds(G + off, 16)]
                        a0s = aos[pl.ds(G + off, 16)]
                        em = plsc.load_gather(
                            lp_buf, [trows, tgt_v[pl.ds(off, 16)]])
                        sk = skip_v[pl.ds(off, 16)]
                    else:
                        a0v = a0s = em = sk = None
                    ld.append([ci, off, shv, shs, e0v, e0s, a0v, a0s, em, sk])
                odd = [x for x in ld if x[0] < NCHUNK_O]
                v2 = [x[2] + x[9] for x in odd]
                p1 = [jnp.maximum(x[6], x[4]) for x in odd]
                p = [jnp.maximum(a, b) for a, b in zip(p1, v2)]
                e1 = [jnp.exp(x[6] - q) for x, q in zip(odd, p)]
                e2 = [jnp.exp(x[4] - q) for x, q in zip(odd, p)]
                e3 = [jnp.exp(a - q) for a, q in zip(v2, p)]
                t1 = [a * x[7] for a, x in zip(e1, odd)]
                t2 = [a * x[5] for a, x in zip(e2, odd)]
                t3 = [a * x[3] for a, x in zip(e3, odd)]
                sn = [a + b + c for a, b, c in zip(t1, t2, t3)]
                ov = [q + x[8] for q, x in zip(p, odd)]
                pe = [jnp.maximum(x[4], x[2]) for x in ld]
                f1 = [jnp.exp(x[4] - q) for x, q in zip(ld, pe)]
                f2 = [jnp.exp(x[2] - q) for x, q in zip(ld, pe)]
                se = [a * x[5] + b * x[3] for a, b, x in zip(f1, f2, ld)]
                ev = [q + blankv for q in pe]
                for j, x in enumerate(odd):
                    aov[pl.ds(G + x[1], 16)] = ov[j]
                    aos[pl.ds(G + x[1], 16)] = sn[j]
                for j, x in enumerate(ld):
                    aev[pl.ds(G + x[1], 16)] = ev[j]
                    aes[pl.ds(G + x[1], 16)] = se[j]

        def fold():
            for ci in range(NCHUNK_O):
                off = G + 16 * ci
                aov[pl.ds(off, 16)] = aov[pl.ds(off, 16)] + _polylog(aos[pl.ds(off, 16)])
                aos[pl.ds(off, 16)] = ones
            for ci in range(NCHUNK_E):
                off = G + 16 * ci
                aev[pl.ds(off, 16)] = aev[pl.ds(off, 16)] + _polylog(aes[pl.ds(off, 16)])
                aes[pl.ds(off, 16)] = ones

        def outer(i, carry):
            for kb in (0, 1):
                k = 2 * i + kb
                sem = sem0 if kb == 0 else sem1
                half = kb * TB
                pltpu.make_async_copy(
                    lp_hbm.at[b, pl.ds(k * TB, TB), :],
                    lp_buf.at[pl.ds(half, TB), :], sem,
                ).wait()

                lo = jnp.maximum(k * TB, 1)
                hi = jnp.maximum(lo, jnp.minimum((k + 1) * TB, il))
                lax.fori_loop(lo, hi, lambda t, c: (one_step(t, k, kb), c)[1],
                              0, unroll=False)
                fold()

                @pl.when(k + 2 < NB)
                def _prefetch():
                    pltpu.make_async_copy(
                        lp_hbm.at[b, pl.ds((k + 2) * TB, TB), :],
                        lp_buf.at[pl.ds(half, TB), :], sem,
                    ).start()

            return carry

        lax.fori_loop(0, NB // 2, outer, 0, unroll=False)

        L = tl_v[pl.ds(0, 16)][0]
        v1 = plsc.load_gather(aev, [zeros + (G + L)])
        v2 = plsc.load_gather(aov, [zeros + (G - 1 + L)])
        m = jnp.maximum(v1, v2)
        s = jnp.exp(v1 - m) + jnp.exp(v2 - m)
        res_v[...] = m + _polylog(s)
        pltpu.sync_copy(res_v, out_hbm.at[b])


@functools.cache
def _sc_num():
  return functools.partial(
    pl.kernel,
    out_type=jax.ShapeDtypeStruct((B, 16), jnp.float32),
    mesh=plsc.VectorSubcoreMesh(core_axis_name="c", subcore_axis_name="s",
                                num_cores=2, num_subcores=16),
    compiler_params=pltpu.CompilerParams(needs_layout_passes=False),
    scratch_types=[
        pltpu.VMEM((2 * TB, C), jnp.float32),
        pltpu.VMEM((C,), jnp.float32),
        pltpu.VMEM((U,), jnp.int32),
        pltpu.VMEM((U,), jnp.float32),
        pltpu.VMEM((16,), jnp.int32),
        pltpu.VMEM((16,), jnp.int32),
        pltpu.VMEM((16,), jnp.float32),
        pltpu.VMEM((ALEN,), jnp.float32),
        pltpu.VMEM((ALEN,), jnp.float32),
        pltpu.VMEM((ALEN,), jnp.float32),
        pltpu.VMEM((ALEN,), jnp.float32),
        pltpu.SemaphoreType.DMA,
        pltpu.SemaphoreType.DMA,
    ],
  )(_sc_num_body)


TBD = 256
NBD = T // TBD


def _den_body(il_ref, num_ref, lp_ref, out_ref, acc_ref):
    i = pl.program_id(0)

    @pl.when(i == 0)
    def _init():
        acc_ref[...] = jnp.zeros_like(acc_ref)

    lp = lp_ref[...]
    mx = jnp.max(lp, axis=2)
    s = jnp.sum(jnp.exp(lp - mx[:, :, None]), axis=2)
    lse = mx + jnp.log(s)
    t = i * TBD + lax.broadcasted_iota(jnp.int32, (B, TBD), 1)
    mask = t < il_ref[:, 0:1]
    acc_ref[...] += jnp.where(mask, lse, 0.0)

    @pl.when(i == NBD - 1)
    def _fin():
        den = jnp.sum(acc_ref[...], axis=1, keepdims=True)
        num = num_ref[:, 0:1]
        tot = num - DEN_SCALE * den
        valid = tot > 0.5 * NEG_INF
        ilf = il_ref[:, 0:1].astype(jnp.float32)
        nf = jnp.sum(jnp.where(valid, ilf, 0.0))
        mmi = jnp.sum(jnp.where(valid, tot, 0.0)) / jnp.maximum(nf, 1.0)
        out_ref[0, 0] = -mmi


_den = pl.pallas_call(
    _den_body,
    grid=(NBD,),
    in_specs=[
        pl.BlockSpec((B, 128), lambda i: (0, 0)),
        pl.BlockSpec((B, 128), lambda i: (0, 0)),
        pl.BlockSpec((B, TBD, C), lambda i: (0, i, 0)),
    ],
    out_specs=pl.BlockSpec((1, 1), lambda i: (0, 0), memory_space=pltpu.SMEM),
    out_shape=jax.ShapeDtypeStruct((1, 1), jnp.float32),
    scratch_shapes=[pltpu.VMEM((B, TBD), jnp.float32)],
)


def kernel(log_probs, targets, input_lengths, target_lengths):
    targets = targets.astype(jnp.int32)
    diff = jnp.concatenate(
        [jnp.zeros((B, 1), bool), targets[:, 1:] != targets[:, :-1]], axis=1)
    skipinf = jnp.where(diff, 0.0, NEG_INF).astype(jnp.float32)
    il16 = jnp.broadcast_to(input_lengths.astype(jnp.int32)[:, None], (B, 16))
    tl16 = jnp.broadcast_to(target_lengths.astype(jnp.int32)[:, None], (B, 16))

    num16 = _sc_num()(log_probs, targets, skipinf, il16, tl16)

    num128 = jnp.broadcast_to(num16[:, 0:1], (B, 128))
    il128 = jnp.broadcast_to(input_lengths.astype(jnp.int32)[:, None], (B, 128))
    loss = _den(il128, num128, log_probs)
    return loss[0, 0]


# speedup vs baseline: 1.1543x; 1.1543x over previous
import functools

import jax
import jax.numpy as jnp
from jax import lax
from jax.experimental import pallas as pl
from jax.experimental.pallas import tpu as pltpu
from jax.experimental.pallas import tpu_sc as plsc

B, T, C, U = 16, 2048, 512, 256
BLANK = 0
DEN_SCALE = 1.0
NEG_INF = -1e30

TB = 64
NB = T // TB
G = 16
NCHUNK_O = U // 16
NCHUNK_E = U // 16 + 1
_GROUPS = ([16, 15, 14, 13, 12, 11], [10, 9, 8, 7, 6, 5], [4, 3, 2, 1, 0])
ALEN = G + U + 16

_LOG_COEF = (
    -0.01720806024968624, 0.18497517704963684, -0.8555376529693604,
    2.2311506271362305, -3.648834466934204, 4.204533100128174,
    -2.0990748405456543,
)
_LN2 = 0.6931471805599453


def _polylog(s):
    bits = plsc.bitcast(s, jnp.int32)
    e = (bits >> 23) - 127
    m = plsc.bitcast((bits & 0x007FFFFF) | 0x3F800000, jnp.float32)
    acc = jnp.full_like(m, _LOG_COEF[0])
    for c in _LOG_COEF[1:]:
        acc = acc * m + c
    return e.astype(jnp.float32) * _LN2 + acc


def _sc_num_body(lp_hbm, tgt_hbm, skip_hbm, il_hbm, tl_hbm, out_hbm,
                 lp_buf, row_v, tgt_v, skip_v, il_v, tl_v, res_v,
                 aov, aos, aev, aes, sem0, sem1):
    wid = lax.axis_index("s") * 2 + lax.axis_index("c")

    @pl.when(wid < B)
    def _worker():
        b = wid
        iota = lax.iota(jnp.int32, 16)
        zeros = iota * 0
        neg = jnp.full((16,), NEG_INF, jnp.float32)
        ones = jnp.full((16,), 1.0, jnp.float32)

        pltpu.sync_copy(tgt_hbm.at[b], tgt_v)
        pltpu.sync_copy(skip_hbm.at[b], skip_v)
        pltpu.sync_copy(il_hbm.at[b], il_v)
        pltpu.sync_copy(tl_hbm.at[b], tl_v)
        pltpu.sync_copy(lp_hbm.at[b, 0], row_v)
        il = il_v[pl.ds(0, 16)][0]

        pltpu.make_async_copy(
            lp_hbm.at[b, pl.ds(0, TB), :], lp_buf.at[pl.ds(0, TB), :], sem0
        ).start()
        pltpu.make_async_copy(
            lp_hbm.at[b, pl.ds(TB, TB), :], lp_buf.at[pl.ds(TB, TB), :], sem1
        ).start()

        for cidx in range(ALEN // 16):
            aov[pl.ds(16 * cidx, 16)] = neg
            aev[pl.ds(16 * cidx, 16)] = neg
            aos[pl.ds(16 * cidx, 16)] = ones
            aes[pl.ds(16 * cidx, 16)] = ones
        blank0 = plsc.load_gather(row_v, [zeros])
        tgt0 = plsc.load_gather(row_v, [tgt_v[pl.ds(0, 16)]])
        first = iota == 0
        aev[pl.ds(G, 16)] = jnp.where(first, blank0, neg)
        aov[pl.ds(G, 16)] = jnp.where(first, tgt0, neg)

        def one_step(t, k, kb):
            trow = (t - k * TB) + kb * TB
            trows = zeros + trow
            blankv = plsc.load_gather(lp_buf, [trows, zeros])
            for grp in _GROUPS:
                ld = []
                for ci in grp:
                    off = 16 * ci
                    idx = iota + (G - 1 + off)
                    shv = plsc.load_gather(aov, [idx])
                    shs = plsc.load_gather(aos, [idx])
                    e0v = aev[pl.ds(G + off, 16)]
                    e0s = aes[pl.ds(G + off, 16)]
                    if ci < NCHUNK_O:
                        a0v = aov[pl.ds(G + off, 16)]
                        a0s = aos[pl.ds(G + off, 16)]
                        em = plsc.load_gather(
                            lp_buf, [trows, tgt_v[pl.ds(off, 16)]])
                        sk = skip_v[pl.ds(off, 16)]
                    else:
                        a0v = a0s = em = sk = None
                    ld.append([ci, off, shv, shs, e0v, e0s, a0v, a0s, em, sk])
                odd = [x for x in ld if x[0] < NCHUNK_O]
                v2 = [x[2] + x[9] for x in odd]
                p1 = [jnp.maximum(x[6], x[4]) for x in odd]
                p = [jnp.maximum(a, b) for a, b in zip(p1, v2)]
                e1 = [jnp.exp(x[6] - q) for x, q in zip(odd, p)]
                e2 = [jnp.exp(x[4] - q) for x, q in zip(odd, p)]
                e3 = [jnp.exp(a - q) for a, q in zip(v2, p)]
                t1 = [a * x[7] for a, x in zip(e1, odd)]
                t2 = [a * x[5] for a, x in zip(e2, odd)]
                t3 = [a * x[3] for a, x in zip(e3, odd)]
                sn = [a + b + c for a, b, c in zip(t1, t2, t3)]
                ov = [q + x[8] for q, x in zip(p, odd)]
                pe = [jnp.maximum(x[4], x[2]) for x in ld]
                d = [x[4] - x[2] for x in ld]
                ft = [jnp.exp(jnp.minimum(q, -q)) for q in d]
                se = [jnp.where(q >= 0, x[5] + a * x[3], x[3] + a * x[5])
                      for q, a, x in zip(d, ft, ld)]
                ev = [q + blankv for q in pe]
                for j, x in enumerate(odd):
                    aov[pl.ds(G + x[1], 16)] = ov[j]
                    aos[pl.ds(G + x[1], 16)] = sn[j]
                for j, x in enumerate(ld):
                    aev[pl.ds(G + x[1], 16)] = ev[j]
                    aes[pl.ds(G + x[1], 16)] = se[j]

        def fold():
            for ci in range(NCHUNK_O):
                off = G + 16 * ci
                aov[pl.ds(off, 16)] = aov[pl.ds(off, 16)] + _polylog(aos[pl.ds(off, 16)])
                aos[pl.ds(off, 16)] = ones
            for ci in range(NCHUNK_E):
                off = G + 16 * ci
                aev[pl.ds(off, 16)] = aev[pl.ds(off, 16)] + _polylog(aes[pl.ds(off, 16)])
                aes[pl.ds(off, 16)] = ones

        def outer(i, carry):
            for kb in (0, 1):
                k = 2 * i + kb
                sem = sem0 if kb == 0 else sem1
                half = kb * TB
                pltpu.make_async_copy(
                    lp_hbm.at[b, pl.ds(k * TB, TB), :],
                    lp_buf.at[pl.ds(half, TB), :], sem,
                ).wait()

                lo = jnp.maximum(k * TB, 1)
                hi = jnp.maximum(lo, jnp.minimum((k + 1) * TB, il))
                lax.fori_loop(lo, hi, lambda t, c: (one_step(t, k, kb), c)[1],
                              0, unroll=False)
                fold()

                @pl.when(k + 2 < NB)
                def _prefetch():
                    pltpu.make_async_copy(
                        lp_hbm.at[b, pl.ds((k + 2) * TB, TB), :],
                        lp_buf.at[pl.ds(half, TB), :], sem,
                    ).start()

            return carry

        lax.fori_loop(0, NB // 2, outer, 0, unroll=False)

        L = tl_v[pl.ds(0, 16)][0]
        v1 = plsc.load_gather(aev, [zeros + (G + L)])
        v2 = plsc.load_gather(aov, [zeros + (G - 1 + L)])
        m = jnp.maximum(v1, v2)
        s = jnp.exp(v1 - m) + jnp.exp(v2 - m)
        res_v[...] = m + _polylog(s)
        pltpu.sync_copy(res_v, out_hbm.at[b])


@functools.cache
def _sc_num():
  return functools.partial(
    pl.kernel,
    out_type=jax.ShapeDtypeStruct((B, 16), jnp.float32),
    mesh=plsc.VectorSubcoreMesh(core_axis_name="c", subcore_axis_name="s",
                                num_cores=2, num_subcores=16),
    compiler_params=pltpu.CompilerParams(needs_layout_passes=False),
    scratch_types=[
        pltpu.VMEM((2 * TB, C), jnp.float32),
        pltpu.VMEM((C,), jnp.float32),
        pltpu.VMEM((U,), jnp.int32),
        pltpu.VMEM((U,), jnp.float32),
        pltpu.VMEM((16,), jnp.int32),
        pltpu.VMEM((16,), jnp.int32),
        pltpu.VMEM((16,), jnp.float32),
        pltpu.VMEM((ALEN,), jnp.float32),
        pltpu.VMEM((ALEN,), jnp.float32),
        pltpu.VMEM((ALEN,), jnp.float32),
        pltpu.VMEM((ALEN,), jnp.float32),
        pltpu.SemaphoreType.DMA,
        pltpu.SemaphoreType.DMA,
    ],
  )(_sc_num_body)


TBD = 256
NBD = T // TBD


def _den_body(il_ref, num_ref, lp_ref, out_ref, acc_ref):
    i = pl.program_id(0)

    @pl.when(i == 0)
    def _init():
        acc_ref[...] = jnp.zeros_like(acc_ref)

    lp = lp_ref[...]
    mx = jnp.max(lp, axis=2)
    s = jnp.sum(jnp.exp(lp - mx[:, :, None]), axis=2)
    lse = mx + jnp.log(s)
    t = i * TBD + lax.broadcasted_iota(jnp.int32, (B, TBD), 1)
    mask = t < il_ref[:, 0:1]
    acc_ref[...] += jnp.where(mask, lse, 0.0)

    @pl.when(i == NBD - 1)
    def _fin():
        den = jnp.sum(acc_ref[...], axis=1, keepdims=True)
        num = num_ref[:, 0:1]
        tot = num - DEN_SCALE * den
        valid = tot > 0.5 * NEG_INF
        ilf = il_ref[:, 0:1].astype(jnp.float32)
        nf = jnp.sum(jnp.where(valid, ilf, 0.0))
        mmi = jnp.sum(jnp.where(valid, tot, 0.0)) / jnp.maximum(nf, 1.0)
        out_ref[0, 0] = -mmi


_den = pl.pallas_call(
    _den_body,
    grid=(NBD,),
    in_specs=[
        pl.BlockSpec((B, 128), lambda i: (0, 0)),
        pl.BlockSpec((B, 128), lambda i: (0, 0)),
        pl.BlockSpec((B, TBD, C), lambda i: (0, i, 0)),
    ],
    out_specs=pl.BlockSpec((1, 1), lambda i: (0, 0), memory_space=pltpu.SMEM),
    out_shape=jax.ShapeDtypeStruct((1, 1), jnp.float32),
    scratch_shapes=[pltpu.VMEM((B, TBD), jnp.float32)],
)


def kernel(log_probs, targets, input_lengths, target_lengths):
    targets = targets.astype(jnp.int32)
    diff = jnp.concatenate(
        [jnp.zeros((B, 1), bool), targets[:, 1:] != targets[:, :-1]], axis=1)
    skipinf = jnp.where(diff, 0.0, NEG_INF).astype(jnp.float32)
    il16 = jnp.broadcast_to(input_lengths.astype(jnp.int32)[:, None], (B, 16))
    tl16 = jnp.broadcast_to(target_lengths.astype(jnp.int32)[:, None], (B, 16))

    num16 = _sc_num()(log_probs, targets, skipinf, il16, tl16)

    num128 = jnp.broadcast_to(num16[:, 0:1], (B, 128))
    il128 = jnp.broadcast_to(input_lengths.astype(jnp.int32)[:, None], (B, 128))
    loss = _den(il128, num128, log_probs)
    return loss[0, 0]


# speedup vs baseline: 1.2082x; 1.0467x over previous
import functools

import jax
import jax.numpy as jnp
from jax import lax
from jax.experimental import pallas as pl
from jax.experimental.pallas import tpu as pltpu
from jax.experimental.pallas import tpu_sc as plsc

B, T, C, U = 16, 2048, 512, 256
BLANK = 0
DEN_SCALE = 1.0
NEG_INF = -1e30

TB = 64
NB = T // TB
G = 16
NCHUNK_O = U // 16
NCHUNK_E = U // 16 + 1
_GROUPS = ([16, 15, 14, 13], [12, 11, 10, 9], [8, 7, 6, 5], [4, 3, 2, 1], [0])
ALEN = G + U + 16

_LOG_COEF = (
    -0.01720806024968624, 0.18497517704963684, -0.8555376529693604,
    2.2311506271362305, -3.648834466934204, 4.204533100128174,
    -2.0990748405456543,
)
_LN2 = 0.6931471805599453


def _polylog(s):
    bits = plsc.bitcast(s, jnp.int32)
    e = (bits >> 23) - 127
    m = plsc.bitcast((bits & 0x007FFFFF) | 0x3F800000, jnp.float32)
    acc = jnp.full_like(m, _LOG_COEF[0])
    for c in _LOG_COEF[1:]:
        acc = acc * m + c
    return e.astype(jnp.float32) * _LN2 + acc


def _sc_num_body(lp_hbm, tgt_hbm, skip_hbm, il_hbm, tl_hbm, out_hbm,
                 lp_buf, row_v, tgt_v, skip_v, il_v, tl_v, res_v,
                 aov, aos, aev, aes, sem0, sem1):
    wid = lax.axis_index("s") * 2 + lax.axis_index("c")

    @pl.when(wid < B)
    def _worker():
        b = wid
        iota = lax.iota(jnp.int32, 16)
        zeros = iota * 0
        neg = jnp.full((16,), NEG_INF, jnp.float32)
        ones = jnp.full((16,), 1.0, jnp.float32)

        pltpu.sync_copy(tgt_hbm.at[b], tgt_v)
        pltpu.sync_copy(skip_hbm.at[b], skip_v)
        pltpu.sync_copy(il_hbm.at[b], il_v)
        pltpu.sync_copy(tl_hbm.at[b], tl_v)
        pltpu.sync_copy(lp_hbm.at[b, 0], row_v)
        il = il_v[pl.ds(0, 16)][0]

        pltpu.make_async_copy(
            lp_hbm.at[b, pl.ds(0, TB), :], lp_buf.at[pl.ds(0, TB), :], sem0
        ).start()
        pltpu.make_async_copy(
            lp_hbm.at[b, pl.ds(TB, TB), :], lp_buf.at[pl.ds(TB, TB), :], sem1
        ).start()

        for cidx in range(ALEN // 16):
            aov[pl.ds(16 * cidx, 16)] = neg
            aev[pl.ds(16 * cidx, 16)] = neg
            aos[pl.ds(16 * cidx, 16)] = ones
            aes[pl.ds(16 * cidx, 16)] = ones
        blank0 = plsc.load_gather(row_v, [zeros])
        tgt0 = plsc.load_gather(row_v, [tgt_v[pl.ds(0, 16)]])
        first = iota == 0
        aev[pl.ds(G, 16)] = jnp.where(first, blank0, neg)
        aov[pl.ds(G, 16)] = jnp.where(first, tgt0, neg)

        def one_step(t, k, kb):
            trow = (t - k * TB) + kb * TB
            trows = zeros + trow
            blankv = plsc.load_gather(lp_buf, [trows, zeros])
            def emit_loads(grp):
                ld = []
                for ci in grp:
                    off = 16 * ci
                    idx = iota + (G - 1 + off)
                    shv = plsc.load_gather(aov, [idx])
                    shs = plsc.load_gather(aos, [idx])
                    e0v = aev[pl.ds(G + off, 16)]
                    e0s = aes[pl.ds(G + off, 16)]
                    if ci < NCHUNK_O:
                        a0v = aov[pl.ds(G + off, 16)]
                        a0s = aos[pl.ds(G + off, 16)]
                        em = plsc.load_gather(
                            lp_buf, [trows, tgt_v[pl.ds(off, 16)]])
                        sk = skip_v[pl.ds(off, 16)]
                    else:
                        a0v = a0s = em = sk = None
                    ld.append([ci, off, shv, shs, e0v, e0s, a0v, a0s, em, sk])
                return ld

            def emit_compute_store(ld):
                odd = [x for x in ld if x[0] < NCHUNK_O]
                v2 = [x[2] + x[9] for x in odd]
                p1 = [jnp.maximum(x[6], x[4]) for x in odd]
                p = [jnp.maximum(a, b) for a, b in zip(p1, v2)]
                e1 = [jnp.exp(x[6] - q) for x, q in zip(odd, p)]
                e2 = [jnp.exp(x[4] - q) for x, q in zip(odd, p)]
                e3 = [jnp.exp(a - q) for a, q in zip(v2, p)]
                t1 = [a * x[7] for a, x in zip(e1, odd)]
                t2 = [a * x[5] for a, x in zip(e2, odd)]
                t3 = [a * x[3] for a, x in zip(e3, odd)]
                sn = [a + b + c for a, b, c in zip(t1, t2, t3)]
                ov = [q + x[8] for q, x in zip(p, odd)]
                pe = [jnp.maximum(x[4], x[2]) for x in ld]
                d = [x[4] - x[2] for x in ld]
                ft = [jnp.exp(jnp.minimum(q, -q)) for q in d]
                se = [jnp.where(q >= 0, x[5] + a * x[3], x[3] + a * x[5])
                      for q, a, x in zip(d, ft, ld)]
                ev = [q + blankv for q in pe]
                for j, x in enumerate(odd):
                    aov[pl.ds(G + x[1], 16)] = ov[j]
                    aos[pl.ds(G + x[1], 16)] = sn[j]
                for j, x in enumerate(ld):
                    aev[pl.ds(G + x[1], 16)] = ev[j]
                    aes[pl.ds(G + x[1], 16)] = se[j]

            pend = None
            for grp in _GROUPS:
                nxt = emit_loads(grp)
                if pend is not None:
                    emit_compute_store(pend)
                pend = nxt
            emit_compute_store(pend)

        def fold():
            for ci in range(NCHUNK_O):
                off = G + 16 * ci
                aov[pl.ds(off, 16)] = aov[pl.ds(off, 16)] + _polylog(aos[pl.ds(off, 16)])
                aos[pl.ds(off, 16)] = ones
            for ci in range(NCHUNK_E):
                off = G + 16 * ci
                aev[pl.ds(off, 16)] = aev[pl.ds(off, 16)] + _polylog(aes[pl.ds(off, 16)])
                aes[pl.ds(off, 16)] = ones

        def outer(i, carry):
            for kb in (0, 1):
                k = 2 * i + kb
                sem = sem0 if kb == 0 else sem1
                half = kb * TB
                pltpu.make_async_copy(
                    lp_hbm.at[b, pl.ds(k * TB, TB), :],
                    lp_buf.at[pl.ds(half, TB), :], sem,
                ).wait()

                lo = jnp.maximum(k * TB, 1)
                hi = jnp.maximum(lo, jnp.minimum((k + 1) * TB, il))
                lax.fori_loop(lo, hi, lambda t, c: (one_step(t, k, kb), c)[1],
                              0, unroll=False)
                fold()

                @pl.when(k + 2 < NB)
                def _prefetch():
                    pltpu.make_async_copy(
                        lp_hbm.at[b, pl.ds((k + 2) * TB, TB), :],
                        lp_buf.at[pl.ds(half, TB), :], sem,
                    ).start()

            return carry

        lax.fori_loop(0, NB // 2, outer, 0, unroll=False)

        L = tl_v[pl.ds(0, 16)][0]
        v1 = plsc.load_gather(aev, [zeros + (G + L)])
        v2 = plsc.load_gather(aov, [zeros + (G - 1 + L)])
        m = jnp.maximum(v1, v2)
        s = jnp.exp(v1 - m) + jnp.exp(v2 - m)
        res_v[...] = m + _polylog(s)
        pltpu.sync_copy(res_v, out_hbm.at[b])


@functools.cache
def _sc_num():
  return functools.partial(
    pl.kernel,
    out_type=jax.ShapeDtypeStruct((B, 16), jnp.float32),
    mesh=plsc.VectorSubcoreMesh(core_axis_name="c", subcore_axis_name="s",
                                num_cores=2, num_subcores=16),
    compiler_params=pltpu.CompilerParams(needs_layout_passes=False),
    scratch_types=[
        pltpu.VMEM((2 * TB, C), jnp.float32),
        pltpu.VMEM((C,), jnp.float32),
        pltpu.VMEM((U,), jnp.int32),
        pltpu.VMEM((U,), jnp.float32),
        pltpu.VMEM((16,), jnp.int32),
        pltpu.VMEM((16,), jnp.int32),
        pltpu.VMEM((16,), jnp.float32),
        pltpu.VMEM((ALEN,), jnp.float32),
        pltpu.VMEM((ALEN,), jnp.float32),
        pltpu.VMEM((ALEN,), jnp.float32),
        pltpu.VMEM((ALEN,), jnp.float32),
        pltpu.SemaphoreType.DMA,
        pltpu.SemaphoreType.DMA,
    ],
  )(_sc_num_body)


TBD = 256
NBD = T // TBD


def _den_body(il_ref, num_ref, lp_ref, out_ref, acc_ref):
    i = pl.program_id(0)

    @pl.when(i == 0)
    def _init():
        acc_ref[...] = jnp.zeros_like(acc_ref)

    lp = lp_ref[...]
    mx = jnp.max(lp, axis=2)
    s = jnp.sum(jnp.exp(lp - mx[:, :, None]), axis=2)
    lse = mx + jnp.log(s)
    t = i * TBD + lax.broadcasted_iota(jnp.int32, (B, TBD), 1)
    mask = t < il_ref[:, 0:1]
    acc_ref[...] += jnp.where(mask, lse, 0.0)

    @pl.when(i == NBD - 1)
    def _fin():
        den = jnp.sum(acc_ref[...], axis=1, keepdims=True)
        num = num_ref[:, 0:1]
        tot = num - DEN_SCALE * den
        valid = tot > 0.5 * NEG_INF
        ilf = il_ref[:, 0:1].astype(jnp.float32)
        nf = jnp.sum(jnp.where(valid, ilf, 0.0))
        mmi = jnp.sum(jnp.where(valid, tot, 0.0)) / jnp.maximum(nf, 1.0)
        out_ref[0, 0] = -mmi


_den = pl.pallas_call(
    _den_body,
    grid=(NBD,),
    in_specs=[
        pl.BlockSpec((B, 128), lambda i: (0, 0)),
        pl.BlockSpec((B, 128), lambda i: (0, 0)),
        pl.BlockSpec((B, TBD, C), lambda i: (0, i, 0)),
    ],
    out_specs=pl.BlockSpec((1, 1), lambda i: (0, 0), memory_space=pltpu.SMEM),
    out_shape=jax.ShapeDtypeStruct((1, 1), jnp.float32),
    scratch_shapes=[pltpu.VMEM((B, TBD), jnp.float32)],
)


def kernel(log_probs, targets, input_lengths, target_lengths):
    targets = targets.astype(jnp.int32)
    diff = jnp.concatenate(
        [jnp.zeros((B, 1), bool), targets[:, 1:] != targets[:, :-1]], axis=1)
    skipinf = jnp.where(diff, 0.0, NEG_INF).astype(jnp.float32)
    il16 = jnp.broadcast_to(input_lengths.astype(jnp.int32)[:, None], (B, 16))
    tl16 = jnp.broadcast_to(target_lengths.astype(jnp.int32)[:, None], (B, 16))

    num16 = _sc_num()(log_probs, targets, skipinf, il16, tl16)

    num128 = jnp.broadcast_to(num16[:, 0:1], (B, 128))
    il128 = jnp.broadcast_to(input_lengths.astype(jnp.int32)[:, None], (B, 128))
    loss = _den(il128, num128, log_probs)
    return loss[0, 0]
